# baseline (speedup 1.0000x reference)
import functools

import jax
import jax.numpy as jnp
from jax import lax
from jax.experimental import pallas as pl
from jax.experimental.pallas import tpu as pltpu
from jax.experimental.pallas import tpu_sc as plsc

_TILE = 4096
_CHUNK = 1024
_NC = _TILE // _CHUNK
_NW = 32


def _mbox(conf_ref, lab_ref, out_ref, score_ref, acc_ref, *, n, num_tiles):
    i = pl.program_id(0)

    lane = jax.lax.broadcasted_iota(jnp.int32, (1, _CHUNK), 1)
    cls_t = jax.lax.broadcasted_iota(jnp.int32, (81, _CHUNK), 0)

    npv = jnp.zeros((1, _CHUNK), jnp.float32)
    cpv = jnp.zeros((1, _CHUNK), jnp.float32)

    for j in range(_NC):
        sl = pl.ds(j * _CHUNK, _CHUNK)
        base = i * _TILE + j * _CHUNK
        validt = (base + lane) < n
        labt = lab_ref[0, pl.ds(j, 1), :]
        post = validt & (labt > 0)

        conf_t = jnp.transpose(conf_ref[sl, :])
        s = jnp.sum(jnp.exp(conf_t), axis=0, keepdims=True)
        g = jnp.sum(jnp.where(cls_t == labt, conf_t, 0.0),
                    axis=0, keepdims=True)
        ce = jnp.log(s) - g

        score_ref[pl.ds(i * _NC + j, 1), :] = (
            jnp.where(validt, jnp.where(post, 0.0, ce), -jnp.inf))

        npv += post.astype(jnp.float32)
        cpv += jnp.where(post, ce, 0.0)

    @pl.when(i == 0)
    def _init():
        acc_ref[...] = jnp.zeros_like(acc_ref)

    acc_ref[0:1, :] += npv
    acc_ref[1:2, :] += cpv

    @pl.when(i == num_tiles - 1)
    def _select():
        npos = jnp.sum(acc_ref[0:1, :])
        cep = jnp.sum(acc_ref[1:2, :])

        sc = score_ref[...]
        bits = jax.lax.bitcast_convert_type(sc, jnp.int32)
        ukey_i = jnp.where(bits < 0, ~bits, bits ^ jnp.int32(-2147483648))
        ukey = jax.lax.bitcast_convert_type(ukey_i, jnp.uint32)

        num_neg = jnp.minimum(3.0 * npos, jnp.float32(n - 1))
        k = num_neg.astype(jnp.int32)

        def body(_, carry):
            lo, hi = carry
            span = hi - lo
            mid = lo + span // jnp.uint32(2) + (span & jnp.uint32(1))
            ge = jnp.sum((ukey >= mid).astype(jnp.int32)) >= k
            return (jnp.where(ge, mid, lo),
                    jnp.where(ge, hi, mid - jnp.uint32(1)))

        t, _ = jax.lax.fori_loop(
            0, 32, body, (jnp.uint32(0), jnp.uint32(0xFFFFFFFF)))

        gt = ukey > t
        c_gt = jnp.sum(gt.astype(jnp.int32))
        s_gt = jnp.sum(jnp.where(gt, sc, 0.0))
        r = (k - c_gt).astype(jnp.float32)
        t_i = jax.lax.bitcast_convert_type(t, jnp.int32)
        tb = jnp.where(t_i < 0, t_i ^ jnp.int32(-2147483648), ~t_i)
        t_val = jax.lax.bitcast_convert_type(tb, jnp.float32)
        loss_c_sum = cep + s_gt + jnp.where(r > 0, r * t_val, 0.0)

        ri = jax.lax.broadcasted_iota(jnp.int32, (8, 128), 0)
        ci = jax.lax.broadcasted_iota(jnp.int32, (8, 128), 1)
        row0 = ri == 0
        out_ref[...] = (
            jnp.where(row0 & (ci == 0), npos, 0.0)
            + jnp.where(row0 & (ci == 1), loss_c_sum / npos, 0.0))


def _sc_l1(loc_hbm, rois_hbm, lab4_hbm, out_hbm, loc_v, rois_v, lab4_v, acc_v,
           *, ppw):
    wid = lax.axis_index("s") * 2 + lax.axis_index("c")
    cpw = ppw * 4
    pltpu.sync_copy(loc_hbm.at[pl.ds(wid * cpw, cpw)], loc_v)
    pltpu.sync_copy(rois_hbm.at[pl.ds(wid * cpw, cpw)], rois_v)
    pltpu.sync_copy(lab4_hbm.at[pl.ds(wid * cpw, cpw)], lab4_v)

    def body(g, acc):
        sl = pl.ds(g * 16, 16)
        d = loc_v[sl] - rois_v[sl]
        a = jnp.abs(d)
        l1 = jnp.where(a < 1.0, 0.5 * d * d, a - 0.5)
        return acc + jnp.where(lab4_v[sl] > 0, l1, 0.0)

    acc = jax.lax.fori_loop(0, cpw // 16, body,
                            jnp.zeros((16,), jnp.float32))
    acc_v[...] = acc
    pltpu.sync_copy(acc_v, out_hbm.at[pl.ds(wid * 16, 16)])


def kernel(loc_pred, conf_pred, rois, labels):
    n, c = conf_pred.shape
    num_tiles = (n + _TILE - 1) // _TILE
    nchunks = num_tiles * _NC
    npad = nchunks * _CHUNK - n
    np_pad = nchunks * _CHUNK
    ppw = np_pad // _NW

    lab_flat = jnp.pad(labels.astype(jnp.int32), (0, npad))
    lab_lane = lab_flat.reshape(num_tiles, _NC, _CHUNK)
    loc_flat = jnp.pad(loc_pred.reshape(-1), (0, 4 * npad))
    rois_flat = jnp.pad(rois.reshape(-1), (0, 4 * npad))
    lab4_flat = jnp.repeat(lab_flat, 4)

    l1p = pl.kernel(
        functools.partial(_sc_l1, ppw=ppw),
        out_type=jax.ShapeDtypeStruct((_NW * 16,), jnp.float32),
        scratch_types=[
            pltpu.VMEM((ppw * 4,), jnp.float32),
            pltpu.VMEM((ppw * 4,), jnp.float32),
            pltpu.VMEM((ppw * 4,), jnp.int32),
            pltpu.VMEM((16,), jnp.float32),
        ],
        mesh=plsc.VectorSubcoreMesh(core_axis_name="c", subcore_axis_name="s"),
    )(loc_flat, rois_flat, lab4_flat)

    out = pl.pallas_call(
        functools.partial(_mbox, n=n, num_tiles=num_tiles),
        grid=(num_tiles,),
        in_specs=[
            pl.BlockSpec((_TILE, c), lambda i: (i, 0)),
            pl.BlockSpec((1, _NC, _CHUNK), lambda i: (i, 0, 0)),
        ],
        out_specs=pl.BlockSpec((8, 128), lambda i: (0, 0)),
        out_shape=jax.ShapeDtypeStruct((8, 128), jnp.float32),
        scratch_shapes=[
            pltpu.VMEM((nchunks, _CHUNK), jnp.float32),
            pltpu.VMEM((8, _CHUNK), jnp.float32),
        ],
        compiler_params=pltpu.CompilerParams(
            dimension_semantics=("arbitrary",)),
    )(conf_pred, lab_lane)

    npos = out[0, 0]
    return (jnp.sum(l1p) / npos, out[0, 1])


# speedup vs baseline: 4.0016x; 4.0016x over previous
import functools

import jax
import jax.numpy as jnp
from jax.experimental import pallas as pl
from jax.experimental.pallas import tpu as pltpu

_TILE = 5120
_CHUNK = 1024
_NC = _TILE // _CHUNK


def _mbox(loc_ref, conf_ref, rois_ref, lab_ref, out_ref, score_ref, acc_ref,
          *, n, num_tiles):
    i = pl.program_id(0)

    lane = jax.lax.broadcasted_iota(jnp.int32, (1, _CHUNK), 1)
    cls_t = jax.lax.broadcasted_iota(jnp.int32, (81, _CHUNK), 0)

    l1v = jnp.zeros((1, _CHUNK), jnp.float32)
    npv = jnp.zeros((1, _CHUNK), jnp.float32)
    cpv = jnp.zeros((1, _CHUNK), jnp.float32)

    for j in range(_NC):
        sl = pl.ds(j * _CHUNK, _CHUNK)
        base = i * _TILE + j * _CHUNK
        validt = (base + lane) < n
        labt = lab_ref[0, pl.ds(j, 1), :]
        post = validt & (labt > 0)
        posft = post.astype(jnp.float32)

        d = loc_ref[:, sl] - rois_ref[:, sl]
        a = jnp.abs(d)
        l1 = jnp.where(a < 1.0, 0.5 * d * d, a - 0.5)
        l1v += jnp.where(post, jnp.sum(l1, axis=0, keepdims=True), 0.0)

        conf_t = jnp.transpose(conf_ref[sl, :])
        s = jnp.sum(jnp.exp(conf_t), axis=0, keepdims=True)
        g = jnp.sum(jnp.where(cls_t == labt, conf_t, 0.0),
                    axis=0, keepdims=True)
        ce = jnp.log(s) - g

        score_ref[pl.ds(i * _NC + j, 1), :] = (
            jnp.where(validt, jnp.where(post, 0.0, ce), -jnp.inf))

        npv += posft
        cpv += jnp.where(post, ce, 0.0)

    @pl.when(i == 0)
    def _init():
        acc_ref[...] = jnp.zeros_like(acc_ref)

    acc_ref[0:1, :] += l1v
    acc_ref[1:2, :] += npv
    acc_ref[2:3, :] += cpv

    @pl.when(i == num_tiles - 1)
    def _select():
        l1s = jnp.sum(acc_ref[0:1, :])
        npos = jnp.sum(acc_ref[1:2, :])
        cep = jnp.sum(acc_ref[2:3, :])

        sc = score_ref[...]
        bits = jax.lax.bitcast_convert_type(sc, jnp.int32)
        ukey_i = jnp.where(bits < 0, ~bits, bits ^ jnp.int32(-2147483648))
        ukey = jax.lax.bitcast_convert_type(ukey_i, jnp.uint32)

        num_neg = jnp.minimum(3.0 * npos, jnp.float32(n - 1))
        k = num_neg.astype(jnp.int32)

        def body(_, carry):
            lo, hi = carry
            span = hi - lo
            mid = lo + span // jnp.uint32(2) + (span & jnp.uint32(1))
            ge = jnp.sum((ukey >= mid).astype(jnp.int32)) >= k
            return (jnp.where(ge, mid, lo),
                    jnp.where(ge, hi, mid - jnp.uint32(1)))

        t, _ = jax.lax.fori_loop(
            0, 32, body, (jnp.uint32(0), jnp.uint32(0xFFFFFFFF)))

        gt = ukey > t
        c_gt = jnp.sum(gt.astype(jnp.int32))
        s_gt = jnp.sum(jnp.where(gt, sc, 0.0))
        r = (k - c_gt).astype(jnp.float32)
        t_i = jax.lax.bitcast_convert_type(t, jnp.int32)
        tb = jnp.where(t_i < 0, t_i ^ jnp.int32(-2147483648), ~t_i)
        t_val = jax.lax.bitcast_convert_type(tb, jnp.float32)
        loss_c_sum = cep + s_gt + jnp.where(r > 0, r * t_val, 0.0)

        ri = jax.lax.broadcasted_iota(jnp.int32, (8, 128), 0)
        ci = jax.lax.broadcasted_iota(jnp.int32, (8, 128), 1)
        row0 = ri == 0
        out_ref[...] = (
            jnp.where(row0 & (ci == 0), l1s / npos, 0.0)
            + jnp.where(row0 & (ci == 1), loss_c_sum / npos, 0.0))


def kernel(loc_pred, conf_pred, rois, labels):
    n, c = conf_pred.shape
    num_tiles = (n + _TILE - 1) // _TILE
    nchunks = num_tiles * _NC
    npad = nchunks * _CHUNK - n

    lab_lane = jnp.pad(labels.astype(jnp.int32), (0, npad)).reshape(
        num_tiles, _NC, _CHUNK)
    loc_t = jnp.transpose(loc_pred)
    rois_t = jnp.transpose(rois)

    out = pl.pallas_call(
        functools.partial(_mbox, n=n, num_tiles=num_tiles),
        grid=(num_tiles,),
        in_specs=[
            pl.BlockSpec((4, _TILE), lambda i: (0, i)),
            pl.BlockSpec((_TILE, c), lambda i: (i, 0)),
            pl.BlockSpec((4, _TILE), lambda i: (0, i)),
            pl.BlockSpec((1, _NC, _CHUNK), lambda i: (i, 0, 0)),
        ],
        out_specs=pl.BlockSpec((8, 128), lambda i: (0, 0)),
        out_shape=jax.ShapeDtypeStruct((8, 128), jnp.float32),
        scratch_shapes=[
            pltpu.VMEM((nchunks, _CHUNK), jnp.float32),
            pltpu.VMEM((8, _CHUNK), jnp.float32),
        ],
        compiler_params=pltpu.CompilerParams(
            dimension_semantics=("arbitrary",)),
    )(loc_t, conf_pred, rois_t, lab_lane)

    return (out[0, 0], out[0, 1])


# speedup vs baseline: 4.2669x; 1.0663x over previous
import functools

import jax
import jax.numpy as jnp
from jax.experimental import pallas as pl
from jax.experimental.pallas import tpu as pltpu

_TILE = 10240
_CHUNK = 1024
_NC = _TILE // _CHUNK


def _mbox(loc_ref, conf_ref, rois_ref, lab_ref, out_ref, score_ref, acc_ref,
          *, n, num_tiles):
    i = pl.program_id(0)

    lane = jax.lax.broadcasted_iota(jnp.int32, (1, _CHUNK), 1)
    cls_t = jax.lax.broadcasted_iota(jnp.int32, (81, _CHUNK), 0)

    l1v = jnp.zeros((1, _CHUNK), jnp.float32)
    npv = jnp.zeros((1, _CHUNK), jnp.float32)
    cpv = jnp.zeros((1, _CHUNK), jnp.float32)

    for j in range(_NC):
        sl = pl.ds(j * _CHUNK, _CHUNK)
        base = i * _TILE + j * _CHUNK
        validt = (base + lane) < n
        labt = lab_ref[0, pl.ds(j, 1), :]
        post = validt & (labt > 0)
        posft = post.astype(jnp.float32)

        d = loc_ref[:, sl] - rois_ref[:, sl]
        a = jnp.abs(d)
        l1 = jnp.where(a < 1.0, 0.5 * d * d, a - 0.5)
        l1v += jnp.where(post, jnp.sum(l1, axis=0, keepdims=True), 0.0)

        conf_t = jnp.transpose(conf_ref[sl, :])
        s = jnp.sum(jnp.exp(conf_t), axis=0, keepdims=True)
        g = jnp.sum(jnp.where(cls_t == labt, conf_t, 0.0),
                    axis=0, keepdims=True)
        ce = jnp.log(s) - g

        score_ref[pl.ds(i * _NC + j, 1), :] = (
            jnp.where(validt, jnp.where(post, 0.0, ce), -jnp.inf))

        npv += posft
        cpv += jnp.where(post, ce, 0.0)

    @pl.when(i == 0)
    def _init():
        acc_ref[...] = jnp.zeros_like(acc_ref)

    acc_ref[0:1, :] += l1v
    acc_ref[1:2, :] += npv
    acc_ref[2:3, :] += cpv

    @pl.when(i == num_tiles - 1)
    def _select():
        l1s = jnp.sum(acc_ref[0:1, :])
        npos = jnp.sum(acc_ref[1:2, :])
        cep = jnp.sum(acc_ref[2:3, :])

        sc = score_ref[...]
        bits = jax.lax.bitcast_convert_type(sc, jnp.int32)
        ukey_i = jnp.where(bits < 0, ~bits, bits ^ jnp.int32(-2147483648))
        ukey = jax.lax.bitcast_convert_type(ukey_i, jnp.uint32)

        num_neg = jnp.minimum(3.0 * npos, jnp.float32(n - 1))
        k = num_neg.astype(jnp.int32)

        def body(_, carry):
            lo, hi = carry
            span = hi - lo
            mid = lo + span // jnp.uint32(2) + (span & jnp.uint32(1))
            ge = jnp.sum((ukey >= mid).astype(jnp.int32)) >= k
            return (jnp.where(ge, mid, lo),
                    jnp.where(ge, hi, mid - jnp.uint32(1)))

        t, _ = jax.lax.fori_loop(
            0, 32, body, (jnp.uint32(0), jnp.uint32(0xFFFFFFFF)))

        gt = ukey > t
        c_gt = jnp.sum(gt.astype(jnp.int32))
        s_gt = jnp.sum(jnp.where(gt, sc, 0.0))
        r = (k - c_gt).astype(jnp.float32)
        t_i = jax.lax.bitcast_convert_type(t, jnp.int32)
        tb = jnp.where(t_i < 0, t_i ^ jnp.int32(-2147483648), ~t_i)
        t_val = jax.lax.bitcast_convert_type(tb, jnp.float32)
        loss_c_sum = cep + s_gt + jnp.where(r > 0, r * t_val, 0.0)

        ri = jax.lax.broadcasted_iota(jnp.int32, (8, 128), 0)
        ci = jax.lax.broadcasted_iota(jnp.int32, (8, 128), 1)
        row0 = ri == 0
        out_ref[...] = (
            jnp.where(row0 & (ci == 0), l1s / npos, 0.0)
            + jnp.where(row0 & (ci == 1), loss_c_sum / npos, 0.0))


def kernel(loc_pred, conf_pred, rois, labels):
    n, c = conf_pred.shape
    num_tiles = (n + _TILE - 1) // _TILE
    nchunks = num_tiles * _NC
    npad = nchunks * _CHUNK - n

    lab_lane = jnp.pad(labels.astype(jnp.int32), (0, npad)).reshape(
        num_tiles, _NC, _CHUNK)
    loc_t = jnp.transpose(loc_pred)
    rois_t = jnp.transpose(rois)

    out = pl.pallas_call(
        functools.partial(_mbox, n=n, num_tiles=num_tiles),
        grid=(num_tiles,),
        in_specs=[
            pl.BlockSpec((4, _TILE), lambda i: (0, i)),
            pl.BlockSpec((_TILE, c), lambda i: (i, 0)),
            pl.BlockSpec((4, _TILE), lambda i: (0, i)),
            pl.BlockSpec((1, _NC, _CHUNK), lambda i: (i, 0, 0)),
        ],
        out_specs=pl.BlockSpec((8, 128), lambda i: (0, 0)),
        out_shape=jax.ShapeDtypeStruct((8, 128), jnp.float32),
        scratch_shapes=[
            pltpu.VMEM((nchunks, _CHUNK), jnp.float32),
            pltpu.VMEM((8, _CHUNK), jnp.float32),
        ],
        compiler_params=pltpu.CompilerParams(
            dimension_semantics=("arbitrary",)),
    )(loc_t, conf_pred, rois_t, lab_lane)

    return (out[0, 0], out[0, 1])


# speedup vs baseline: 4.3477x; 1.0189x over previous
import functools

import jax
import jax.numpy as jnp
from jax.experimental import pallas as pl
from jax.experimental.pallas import tpu as pltpu

_TILE = 20480
_CHUNK = 1024
_NC = _TILE // _CHUNK


def _mbox(loc_ref, conf_ref, rois_ref, lab_ref, out_ref, score_ref, acc_ref,
          *, n, num_tiles):
    i = pl.program_id(0)

    lane = jax.lax.broadcasted_iota(jnp.int32, (1, _CHUNK), 1)
    cls_t = jax.lax.broadcasted_iota(jnp.int32, (81, _CHUNK), 0)

    l1v = jnp.zeros((1, _CHUNK), jnp.float32)
    npv = jnp.zeros((1, _CHUNK), jnp.float32)
    cpv = jnp.zeros((1, _CHUNK), jnp.float32)

    for j in range(_NC):
        sl = pl.ds(j * _CHUNK, _CHUNK)
        base = i * _TILE + j * _CHUNK
        validt = (base + lane) < n
        labt = lab_ref[0, pl.ds(j, 1), :]
        post = validt & (labt > 0)
        posft = post.astype(jnp.float32)

        d = loc_ref[:, sl] - rois_ref[:, sl]
        a = jnp.abs(d)
        l1 = jnp.where(a < 1.0, 0.5 * d * d, a - 0.5)
        l1v += jnp.where(post, jnp.sum(l1, axis=0, keepdims=True), 0.0)

        conf_t = jnp.transpose(conf_ref[sl, :])
        s = jnp.sum(jnp.exp(conf_t), axis=0, keepdims=True)
        g = jnp.sum(jnp.where(cls_t == labt, conf_t, 0.0),
                    axis=0, keepdims=True)
        ce = jnp.log(s) - g

        score_ref[pl.ds(i * _NC + j, 1), :] = (
            jnp.where(validt, jnp.where(post, 0.0, ce), -jnp.inf))

        npv += posft
        cpv += jnp.where(post, ce, 0.0)

    @pl.when(i == 0)
    def _init():
        acc_ref[...] = jnp.zeros_like(acc_ref)

    acc_ref[0:1, :] += l1v
    acc_ref[1:2, :] += npv
    acc_ref[2:3, :] += cpv

    @pl.when(i == num_tiles - 1)
    def _select():
        l1s = jnp.sum(acc_ref[0:1, :])
        npos = jnp.sum(acc_ref[1:2, :])
        cep = jnp.sum(acc_ref[2:3, :])

        sc = score_ref[...]
        bits = jax.lax.bitcast_convert_type(sc, jnp.int32)
        ukey_i = jnp.where(bits < 0, ~bits, bits ^ jnp.int32(-2147483648))
        ukey = jax.lax.bitcast_convert_type(ukey_i, jnp.uint32)

        num_neg = jnp.minimum(3.0 * npos, jnp.float32(n - 1))
        k = num_neg.astype(jnp.int32)

        def body(_, carry):
            lo, hi = carry
            span = hi - lo
            mid = lo + span // jnp.uint32(2) + (span & jnp.uint32(1))
            ge = jnp.sum((ukey >= mid).astype(jnp.int32)) >= k
            return (jnp.where(ge, mid, lo),
                    jnp.where(ge, hi, mid - jnp.uint32(1)))

        t, _ = jax.lax.fori_loop(
            0, 32, body, (jnp.uint32(0), jnp.uint32(0xFFFFFFFF)))

        gt = ukey > t
        c_gt = jnp.sum(gt.astype(jnp.int32))
        s_gt = jnp.sum(jnp.where(gt, sc, 0.0))
        r = (k - c_gt).astype(jnp.float32)
        t_i = jax.lax.bitcast_convert_type(t, jnp.int32)
        tb = jnp.where(t_i < 0, t_i ^ jnp.int32(-2147483648), ~t_i)
        t_val = jax.lax.bitcast_convert_type(tb, jnp.float32)
        loss_c_sum = cep + s_gt + jnp.where(r > 0, r * t_val, 0.0)

        ri = jax.lax.broadcasted_iota(jnp.int32, (8, 128), 0)
        ci = jax.lax.broadcasted_iota(jnp.int32, (8, 128), 1)
        row0 = ri == 0
        out_ref[...] = (
            jnp.where(row0 & (ci == 0), l1s / npos, 0.0)
            + jnp.where(row0 & (ci == 1), loss_c_sum / npos, 0.0))


def kernel(loc_pred, conf_pred, rois, labels):
    n, c = conf_pred.shape
    num_tiles = (n + _TILE - 1) // _TILE
    nchunks = num_tiles * _NC
    npad = nchunks * _CHUNK - n

    lab_lane = jnp.pad(labels.astype(jnp.int32), (0, npad)).reshape(
        num_tiles, _NC, _CHUNK)
    loc_t = jnp.transpose(loc_pred)
    rois_t = jnp.transpose(rois)

    out = pl.pallas_call(
        functools.partial(_mbox, n=n, num_tiles=num_tiles),
        grid=(num_tiles,),
        in_specs=[
            pl.BlockSpec((4, _TILE), lambda i: (0, i)),
            pl.BlockSpec((_TILE, c), lambda i: (i, 0)),
            pl.BlockSpec((4, _TILE), lambda i: (0, i)),
            pl.BlockSpec((1, _NC, _CHUNK), lambda i: (i, 0, 0)),
        ],
        out_specs=pl.BlockSpec((8, 128), lambda i: (0, 0)),
        out_shape=jax.ShapeDtypeStruct((8, 128), jnp.float32),
        scratch_shapes=[
            pltpu.VMEM((nchunks, _CHUNK), jnp.float32),
            pltpu.VMEM((8, _CHUNK), jnp.float32),
        ],
        compiler_params=pltpu.CompilerParams(
            dimension_semantics=("arbitrary",)),
    )(loc_t, conf_pred, rois_t, lab_lane)

    return (out[0, 0], out[0, 1])


# speedup vs baseline: 4.3488x; 1.0003x over previous
import functools

import jax
import jax.numpy as jnp
from jax.experimental import pallas as pl
from jax.experimental.pallas import tpu as pltpu

_TILE = 25600
_CHUNK = 1024
_NC = _TILE // _CHUNK


def _mbox(loc_ref, conf_ref, rois_ref, lab_ref, out_ref, score_ref, acc_ref,
          *, n, num_tiles):
    i = pl.program_id(0)

    lane = jax.lax.broadcasted_iota(jnp.int32, (1, _CHUNK), 1)
    cls_t = jax.lax.broadcasted_iota(jnp.int32, (81, _CHUNK), 0)

    l1v = jnp.zeros((1, _CHUNK), jnp.float32)
    npv = jnp.zeros((1, _CHUNK), jnp.float32)
    cpv = jnp.zeros((1, _CHUNK), jnp.float32)

    for j in range(_NC):
        sl = pl.ds(j * _CHUNK, _CHUNK)
        base = i * _TILE + j * _CHUNK
        validt = (base + lane) < n
        labt = lab_ref[0, pl.ds(j, 1), :]
        post = validt & (labt > 0)
        posft = post.astype(jnp.float32)

        d = loc_ref[:, sl] - rois_ref[:, sl]
        a = jnp.abs(d)
        l1 = jnp.where(a < 1.0, 0.5 * d * d, a - 0.5)
        l1v += jnp.where(post, jnp.sum(l1, axis=0, keepdims=True), 0.0)

        conf_t = jnp.transpose(conf_ref[sl, :])
        s = jnp.sum(jnp.exp(conf_t), axis=0, keepdims=True)
        g = jnp.sum(jnp.where(cls_t == labt, conf_t, 0.0),
                    axis=0, keepdims=True)
        ce = jnp.log(s) - g

        score_ref[pl.ds(i * _NC + j, 1), :] = (
            jnp.where(validt, jnp.where(post, 0.0, ce), -jnp.inf))

        npv += posft
        cpv += jnp.where(post, ce, 0.0)

    @pl.when(i == 0)
    def _init():
        acc_ref[...] = jnp.zeros_like(acc_ref)

    acc_ref[0:1, :] += l1v
    acc_ref[1:2, :] += npv
    acc_ref[2:3, :] += cpv

    @pl.when(i == num_tiles - 1)
    def _select():
        l1s = jnp.sum(acc_ref[0:1, :])
        npos = jnp.sum(acc_ref[1:2, :])
        cep = jnp.sum(acc_ref[2:3, :])

        sc = score_ref[...]
        bits = jax.lax.bitcast_convert_type(sc, jnp.int32)
        ukey_i = jnp.where(bits < 0, ~bits, bits ^ jnp.int32(-2147483648))
        ukey = jax.lax.bitcast_convert_type(ukey_i, jnp.uint32)

        num_neg = jnp.minimum(3.0 * npos, jnp.float32(n - 1))
        k = num_neg.astype(jnp.int32)

        def body(_, carry):
            lo, hi = carry
            span = hi - lo
            mid = lo + span // jnp.uint32(2) + (span & jnp.uint32(1))
            ge = jnp.sum((ukey >= mid).astype(jnp.int32)) >= k
            return (jnp.where(ge, mid, lo),
                    jnp.where(ge, hi, mid - jnp.uint32(1)))

        t, _ = jax.lax.fori_loop(
            0, 32, body, (jnp.uint32(0), jnp.uint32(0xFFFFFFFF)))

        gt = ukey > t
        c_gt = jnp.sum(gt.astype(jnp.int32))
        s_gt = jnp.sum(jnp.where(gt, sc, 0.0))
        r = (k - c_gt).astype(jnp.float32)
        t_i = jax.lax.bitcast_convert_type(t, jnp.int32)
        tb = jnp.where(t_i < 0, t_i ^ jnp.int32(-2147483648), ~t_i)
        t_val = jax.lax.bitcast_convert_type(tb, jnp.float32)
        loss_c_sum = cep + s_gt + jnp.where(r > 0, r * t_val, 0.0)

        ri = jax.lax.broadcasted_iota(jnp.int32, (8, 128), 0)
        ci = jax.lax.broadcasted_iota(jnp.int32, (8, 128), 1)
        row0 = ri == 0
        out_ref[...] = (
            jnp.where(row0 & (ci == 0), l1s / npos, 0.0)
            + jnp.where(row0 & (ci == 1), loss_c_sum / npos, 0.0))


def kernel(loc_pred, conf_pred, rois, labels):
    n, c = conf_pred.shape
    num_tiles = (n + _TILE - 1) // _TILE
    nchunks = num_tiles * _NC
    npad = nchunks * _CHUNK - n

    lab_lane = jnp.pad(labels.astype(jnp.int32), (0, npad)).reshape(
        num_tiles, _NC, _CHUNK)
    loc_t = jnp.transpose(loc_pred)
    rois_t = jnp.transpose(rois)

    out = pl.pallas_call(
        functools.partial(_mbox, n=n, num_tiles=num_tiles),
        grid=(num_tiles,),
        in_specs=[
            pl.BlockSpec((4, _TILE), lambda i: (0, i)),
            pl.BlockSpec((_TILE, c), lambda i: (i, 0)),
            pl.BlockSpec((4, _TILE), lambda i: (0, i)),
            pl.BlockSpec((1, _NC, _CHUNK), lambda i: (i, 0, 0)),
        ],
        out_specs=pl.BlockSpec((8, 128), lambda i: (0, 0)),
        out_shape=jax.ShapeDtypeStruct((8, 128), jnp.float32),
        scratch_shapes=[
            pltpu.VMEM((nchunks, _CHUNK), jnp.float32),
            pltpu.VMEM((8, _CHUNK), jnp.float32),
        ],
        compiler_params=pltpu.CompilerParams(
            dimension_semantics=("arbitrary",)),
    )(loc_t, conf_pred, rois_t, lab_lane)

    return (out[0, 0], out[0, 1])
